# baseline (speedup 1.0000x reference)
import functools

import jax
import jax.numpy as jnp
from jax import lax
from jax.experimental import pallas as pl
from jax.experimental.pallas import tpu as pltpu
from jax.experimental.pallas import tpu_sc as plsc


def kernel(indices, table):
    B, S = indices.shape
    V, D = table.shape
    N = B * S

    info = plsc.get_sparse_core_info()
    NC, NS = info.num_cores, info.num_subcores
    NW = NC * NS
    CHUNK = 128
    SUPER = 2 * CHUNK
    assert N % (NW * SUPER) == 0
    n_chunks = N // (NW * CHUNK)
    n_super = N // (NW * SUPER)
    assert n_super % 2 == 0 and n_super >= 4

    idx3 = indices.reshape(NW, n_chunks, CHUNK)

    mesh = plsc.VectorSubcoreMesh(core_axis_name="c", subcore_axis_name="s")

    @functools.partial(
        pl.kernel,
        mesh=mesh,
        out_type=jax.ShapeDtypeStruct((N, D), jnp.float32),
        scratch_types=[
            pltpu.VMEM((n_chunks, CHUNK), jnp.int32),
            pltpu.VMEM((SUPER, D), jnp.float32),
            pltpu.VMEM((SUPER, D), jnp.float32),
            pltpu.SemaphoreType.DMA,
            pltpu.SemaphoreType.DMA,
            pltpu.SemaphoreType.DMA,
            pltpu.SemaphoreType.DMA,
        ],
    )
    def sc_gather(idx_hbm, table_hbm, out_hbm, idx_v, sb0, sb1,
                  gsem0, gsem1, wsem0, wsem1):
        wid = lax.axis_index("s") * NC + lax.axis_index("c")
        base = wid * (n_chunks * CHUNK)
        pltpu.sync_copy(idx_hbm.at[wid], idx_v)

        def gather(k, half, buf, sem):
            return pltpu.make_async_copy(
                table_hbm.at[idx_v.at[2 * k + half]],
                buf.at[pl.ds(half * CHUNK, CHUNK)], sem)

        def write(k, buf, sem):
            return pltpu.make_async_copy(
                buf, out_hbm.at[pl.ds(base + k * SUPER, SUPER)], sem)

        def fill(k, buf, sem):
            gather(k, 0, buf, sem).start()
            gather(k, 1, buf, sem).start()

        def drain(k, buf, sem):
            gather(k, 0, buf, sem).wait()
            gather(k, 1, buf, sem).wait()

        fill(0, sb0, gsem0)
        fill(1, sb1, gsem1)
        drain(0, sb0, gsem0)
        write(0, sb0, wsem0).start()
        write(0, sb0, wsem0).wait()
        fill(2, sb0, gsem0)
        drain(1, sb1, gsem1)
        write(1, sb1, wsem1).start()

        def body(g, carry):
            k = 2 * g
            write(k - 1, sb1, wsem1).wait()
            fill(k + 1, sb1, gsem1)
            drain(k, sb0, gsem0)
            write(k, sb0, wsem0).start()
            write(k, sb0, wsem0).wait()
            fill(k + 2, sb0, gsem0)
            drain(k + 1, sb1, gsem1)
            write(k + 1, sb1, wsem1).start()
            return carry

        lax.fori_loop(1, n_super // 2 - 1, body, 0)

        k = n_super - 2
        write(k - 1, sb1, wsem1).wait()
        fill(k + 1, sb1, gsem1)
        drain(k, sb0, gsem0)
        write(k, sb0, wsem0).start()
        write(k, sb0, wsem0).wait()
        drain(k + 1, sb1, gsem1)
        write(k + 1, sb1, wsem1).start()
        write(k + 1, sb1, wsem1).wait()

    out = sc_gather(idx3, table)
    return out.reshape(B, S, D)
